# Initial kernel scaffold; baseline (speedup 1.0000x reference)
import jax
import jax.numpy as jnp
from jax.experimental import pallas as pl


def kernel(x, edge_index, edge_attr, W1, b1, g1, be1, W2, b2, g2, be2, W3, b3, g3, be3, W4, b4, g4, be4, W5, b5, g5, be5, fc1_w, fc1_b, fc2_w, fc2_b):
    raise NotImplementedError("write your pallas kernel here")


# speedup vs baseline: 10.3026x; 10.3026x over previous
import functools

import jax
import jax.numpy as jnp
from jax import lax
from jax.experimental import pallas as pl
from jax.experimental.pallas import tpu as pltpu
from jax.experimental.pallas import tpu_sc as plsc

NC = 2
NS = 16
NW = NC * NS
BN_EPS = 1e-5


@functools.lru_cache(maxsize=None)
def _seg_sum_kernel(N, E, D, colsplit=False):
    NPART = NS if colsplit else NW
    EC = E // NPART
    C = 80
    NCH = EC // C
    HN = NC * N if colsplit else N
    RPT = (N // NS) // 8 * 8
    ZC = 104
    REM = N - NS * RPT
    assert EC % C == 0 and RPT % ZC == 0 and D % 16 == 0 and REM % 8 == 0
    assert REM <= ZC

    mesh = plsc.VectorSubcoreMesh(
        core_axis_name="c", subcore_axis_name="s",
        num_cores=NC, num_subcores=NS)

    def body(h_hbm, src_hbm, dst_hbm, w_hbm, out_hbm,
             sidx, didx, ew, rows, zbuf, acc, sem):
        c = lax.axis_index("c")
        s = lax.axis_index("s")
        wid = s if colsplit else c * NS + s
        zero16 = jnp.zeros((16,), jnp.float32)

        def zrow(r, _):
            for j in range(D // 16):
                zbuf[r, pl.ds(j * 16, 16)] = zero16
            return 0
        lax.fori_loop(0, ZC, zrow, 0)
        row0 = s * RPT
        for k in range(RPT // ZC):
            pltpu.sync_copy(zbuf, acc.at[pl.ds(row0 + k * ZC, ZC)])

        @pl.when(s == NS - 1)
        def _():
            pltpu.sync_copy(zbuf.at[pl.ds(0, REM)],
                            acc.at[pl.ds(NS * RPT, REM)])
        plsc.subcore_barrier()

        pltpu.sync_copy(src_hbm.at[wid], sidx)
        pltpu.sync_copy(dst_hbm.at[wid], didx)
        pltpu.sync_copy(w_hbm.at[wid], ew)
        if colsplit:
            cN = c * N

            def offs(g, _):
                for j in range(C // 16):
                    sidx[g, pl.ds(j * 16, 16)] = (
                        sidx[g, pl.ds(j * 16, 16)] + cN)
                return 0
            lax.fori_loop(0, NCH, offs, 0)

        def chunk(i, _):
            pltpu.async_copy(h_hbm.at[sidx.at[i]], rows, sem).wait()

            def mgrp(g, _):
                nv = ew[i, pl.ds(g * 16, 16)]
                for rr in range(16):
                    r = g * 16 + rr
                    sc = nv[rr]
                    for j in range(D // 16):
                        rows[r, pl.ds(j * 16, 16)] = (
                            rows[r, pl.ds(j * 16, 16)] * sc)
                return 0
            lax.fori_loop(0, C // 16, mgrp, 0)
            pltpu.sync_copy(rows, acc.at[didx.at[i]], add=True)
            return 0
        lax.fori_loop(0, NCH, chunk, 0)
        plsc.subcore_barrier()

        for k in range(RPT // ZC):
            r0 = row0 + k * ZC
            pltpu.sync_copy(acc.at[pl.ds(r0, ZC)], zbuf)
            pltpu.sync_copy(zbuf, out_hbm.at[pl.ds(c * N + r0, ZC)])

        @pl.when(s == NS - 1)
        def _():
            r0 = NS * RPT
            pltpu.sync_copy(acc.at[pl.ds(r0, REM)], zbuf.at[pl.ds(0, REM)])
            pltpu.sync_copy(zbuf.at[pl.ds(0, REM)],
                            out_hbm.at[pl.ds(c * N + r0, REM)])

    return pl.kernel(
        body,
        out_type=jax.ShapeDtypeStruct((NC * N, D), jnp.float32),
        mesh=mesh,
        scratch_types=[
            pltpu.VMEM((NCH, C), jnp.int32),
            pltpu.VMEM((NCH, C), jnp.int32),
            pltpu.VMEM((NCH, C), jnp.float32),
            pltpu.VMEM((C, D), jnp.float32),
            pltpu.VMEM((ZC, D), jnp.float32),
            pltpu.VMEM_SHARED((N, D), jnp.float32),
            pltpu.SemaphoreType.DMA,
        ],
        compiler_params=pltpu.CompilerParams(use_tc_tiling_on_sc=False),
        name=f"seg_sum_d{D}_{'col' if colsplit else 'edge'}",
    )


def _bn(t, g, be):
    mu = jnp.mean(t, axis=0, keepdims=True)
    var = jnp.mean((t - mu) ** 2, axis=0, keepdims=True)
    return g * (t - mu) * lax.rsqrt(var + BN_EPS) + be


def _stage_a(N):
    def body(p_ref, x_ref, w1_ref, dis_ref, hp_ref):
        deg = p_ref[0:N, 0:1] + p_ref[N:2 * N, 0:1] + 1.0
        dis = lax.rsqrt(deg)
        dis_ref[...] = dis
        hw = jnp.dot(x_ref[...], w1_ref[...],
                     preferred_element_type=jnp.float32)
        hp_ref[...] = hw * dis
    return body


def _stage_b(N, relu_first):
    def body(p_ref, hp_ref, dis_ref, b_ref, g_ref, be_ref, w_ref, out_ref):
        dis = dis_ref[...]
        y = dis * (p_ref[0:N, :] + p_ref[N:2 * N, :] + hp_ref[...]) + b_ref[...]
        if relu_first:
            h = _bn(jnp.maximum(y, 0.0), g_ref[...], be_ref[...])
        else:
            h = jnp.maximum(_bn(y, g_ref[...], be_ref[...]), 0.0)
        hw = jnp.dot(h, w_ref[...], preferred_element_type=jnp.float32)
        out_ref[...] = hw * dis
    return body


def _stage_c(N):
    def body(agg_ref, hp_ref, dis_ref, b_ref, g_ref, be_ref,
             fc1w_ref, fc1b_ref, fc2w_ref, fc2b_ref, out_ref):
        dis = dis_ref[...]
        y = dis * (agg_ref[...] + hp_ref[...]) + b_ref[...]
        h = jnp.maximum(_bn(y, g_ref[...], be_ref[...]), 0.0)
        z = jnp.maximum(
            jnp.dot(h, fc1w_ref[...], preferred_element_type=jnp.float32)
            + fc1b_ref[...], 0.0)
        out_ref[...] = (jnp.dot(z, fc2w_ref[...],
                                preferred_element_type=jnp.float32)
                        + fc2b_ref[...])
    return body


def _tc_call(body, out_shapes, *args):
    return pl.pallas_call(
        body,
        out_shape=out_shapes,
    )(*args)


def kernel(x, edge_index, edge_attr,
           W1, b1, g1, be1, W2, b2, g2, be2, W3, b3, g3, be3,
           W4, b4, g4, be4, W5, b5, g5, be5,
           fc1_w, fc1_b, fc2_w, fc2_b):
    N = x.shape[0]
    E = edge_index.shape[1]
    EC = E // NW
    C = 80
    NCH = EC // C

    src3 = edge_index[0].reshape(NW, NCH, C)
    dst3 = edge_index[1].reshape(NW, NCH, C)
    w3 = edge_attr.reshape(NW, NCH, C)

    ones16 = jnp.ones((N, 16), jnp.float32)
    degp = _seg_sum_kernel(N, E, 16)(ones16, src3, dst3, w3)

    dis, hp = _tc_call(
        _stage_a(N),
        (jax.ShapeDtypeStruct((N, 1), jnp.float32),
         jax.ShapeDtypeStruct((N, W1.shape[1]), jnp.float32)),
        degp, x, W1)

    layer_params = [
        (b1, g1, be1, W2, True),
        (b2, g2, be2, W3, True),
        (b3, g3, be3, W4, True),
        (b4, g4, be4, W5, False),
    ]
    for (b, g, be, Wn, relu_first) in layer_params:
        D = hp.shape[1]
        p = _seg_sum_kernel(N, E, D)(hp, src3, dst3, w3)
        hp = _tc_call(
            _stage_b(N, relu_first),
            jax.ShapeDtypeStruct((N, Wn.shape[1]), jnp.float32),
            p, hp, dis, b.reshape(1, -1), g.reshape(1, -1),
            be.reshape(1, -1), Wn)

    D = hp.shape[1]
    DH = D // 2
    srcC = edge_index[0].reshape(NS, -1, 80)
    dstC = edge_index[1].reshape(NS, -1, 80)
    wC = edge_attr.reshape(NS, -1, 80)
    h2 = jnp.concatenate([hp[:, :DH], hp[:, DH:]], axis=0)
    p5 = _seg_sum_kernel(N, E, DH, True)(h2, srcC, dstC, wC)
    agg5 = jnp.concatenate([p5[:N], p5[N:]], axis=1)
    out = _tc_call(
        _stage_c(N),
        jax.ShapeDtypeStruct((N, 1), jnp.float32),
        agg5, hp, dis, b5.reshape(1, -1), g5.reshape(1, -1), be5.reshape(1, -1),
        fc1_w, fc1_b.reshape(1, -1), fc2_w, fc2_b.reshape(1, -1))
    return out.reshape(-1)


# speedup vs baseline: 22.7268x; 2.2059x over previous
import functools

import jax
import jax.numpy as jnp
from jax import lax
from jax.experimental import pallas as pl
from jax.experimental.pallas import tpu as pltpu
from jax.experimental.pallas import tpu_sc as plsc

NC = 2
NS = 16
NW = NC * NS
BN_EPS = 1e-5


@functools.lru_cache(maxsize=None)
def _seg_sum_kernel(N, E, D, colsplit=False):
    NPART = NS if colsplit else NW
    EC = E // NPART
    C = 80
    NCH = EC // C
    HN = NC * N if colsplit else N
    RPT = (N // NS) // 8 * 8
    ZC = 104
    REM = N - NS * RPT
    assert EC % C == 0 and RPT % ZC == 0 and D % 16 == 0 and REM % 8 == 0
    assert REM <= ZC

    mesh = plsc.VectorSubcoreMesh(
        core_axis_name="c", subcore_axis_name="s",
        num_cores=NC, num_subcores=NS)

    def body(h_hbm, src_hbm, dst_hbm, w_hbm, out_hbm,
             sidx, didx, ew, gbuf0, gbuf1, sbuf0, sbuf1, zbuf, acc,
             gsem0, gsem1, ssem0, ssem1):
        c = lax.axis_index("c")
        s = lax.axis_index("s")
        wid = s if colsplit else c * NS + s
        zero16 = jnp.zeros((16,), jnp.float32)

        def zrow(r, _):
            for j in range(D // 16):
                zbuf[r, pl.ds(j * 16, 16)] = zero16
            return 0
        lax.fori_loop(0, ZC, zrow, 0)
        row0 = s * RPT
        for k in range(RPT // ZC):
            pltpu.sync_copy(zbuf, acc.at[pl.ds(row0 + k * ZC, ZC)])

        @pl.when(s == NS - 1)
        def _():
            pltpu.sync_copy(zbuf.at[pl.ds(0, REM)],
                            acc.at[pl.ds(NS * RPT, REM)])
        plsc.subcore_barrier()

        pltpu.sync_copy(src_hbm.at[wid], sidx)
        pltpu.sync_copy(dst_hbm.at[wid], didx)
        pltpu.sync_copy(w_hbm.at[wid], ew)
        if colsplit:
            cN = c * N

            def offs(g, _):
                for j in range(C // 16):
                    sidx[g, pl.ds(j * 16, 16)] = (
                        sidx[g, pl.ds(j * 16, 16)] + cN)
                return 0
            lax.fori_loop(0, NCH, offs, 0)

        gbufs = (gbuf0, gbuf1)
        sbufs = (sbuf0, sbuf1)
        gsems = (gsem0, gsem1)
        ssems = (ssem0, ssem1)

        def start_gather(i, b):
            pltpu.async_copy(h_hbm.at[sidx.at[i]], gbufs[b], gsems[b])

        def wait_gather(i, b):
            pltpu.make_async_copy(h_hbm.at[sidx.at[i]], gbufs[b],
                                  gsems[b]).wait()

        def start_scatter(i, b):
            pltpu.async_copy(sbufs[b], acc.at[didx.at[i]], ssems[b], add=True)

        def wait_scatter(i, b):
            pltpu.make_async_copy(sbufs[b], acc.at[didx.at[i]],
                                  ssems[b]).wait()

        def scale(i, b):
            gb, sb = gbufs[b], sbufs[b]

            def mgrp(g, _):
                nv = ew[i, pl.ds(g * 16, 16)]
                for rr in range(16):
                    r = g * 16 + rr
                    sc = nv[rr]
                    for j in range(D // 16):
                        sb[r, pl.ds(j * 16, 16)] = (
                            gb[r, pl.ds(j * 16, 16)] * sc)
                return 0
            lax.fori_loop(0, C // 16, mgrp, 0)

        start_gather(0, 0)
        start_gather(1, 1)

        def pipe(p, _):
            for b in range(2):
                i = 2 * p + b
                wait_gather(i, b)

                @pl.when(i >= 2)
                def _():
                    wait_scatter(i - 2, b)
                scale(i, b)
                start_scatter(i, b)

                @pl.when(i + 2 < NCH)
                def _():
                    start_gather(i + 2, b)
            return 0
        lax.fori_loop(0, NCH // 2, pipe, 0)

        if NCH % 2:
            i = NCH - 1
            wait_gather(i, i % 2)
            wait_scatter(i - 2, i % 2)
            scale(i, i % 2)
            start_scatter(i, i % 2)
        wait_scatter(NCH - 2, (NCH - 2) % 2)
        wait_scatter(NCH - 1, (NCH - 1) % 2)
        plsc.subcore_barrier()

        for k in range(RPT // ZC):
            r0 = row0 + k * ZC
            pltpu.sync_copy(acc.at[pl.ds(r0, ZC)], zbuf)
            pltpu.sync_copy(zbuf, out_hbm.at[pl.ds(c * N + r0, ZC)])

        @pl.when(s == NS - 1)
        def _():
            r0 = NS * RPT
            pltpu.sync_copy(acc.at[pl.ds(r0, REM)], zbuf.at[pl.ds(0, REM)])
            pltpu.sync_copy(zbuf.at[pl.ds(0, REM)],
                            out_hbm.at[pl.ds(c * N + r0, REM)])

    return pl.kernel(
        body,
        out_type=jax.ShapeDtypeStruct((NC * N, D), jnp.float32),
        mesh=mesh,
        scratch_types=[
            pltpu.VMEM((NCH, C), jnp.int32),
            pltpu.VMEM((NCH, C), jnp.int32),
            pltpu.VMEM((NCH, C), jnp.float32),
            pltpu.VMEM((C, D), jnp.float32),
            pltpu.VMEM((C, D), jnp.float32),
            pltpu.VMEM((C, D), jnp.float32),
            pltpu.VMEM((C, D), jnp.float32),
            pltpu.VMEM((ZC, D), jnp.float32),
            pltpu.VMEM_SHARED((N, D), jnp.float32),
            pltpu.SemaphoreType.DMA,
            pltpu.SemaphoreType.DMA,
            pltpu.SemaphoreType.DMA,
            pltpu.SemaphoreType.DMA,
        ],
        compiler_params=pltpu.CompilerParams(use_tc_tiling_on_sc=False),
        name=f"seg_sum_d{D}_{'col' if colsplit else 'edge'}",
    )


def _bn(t, g, be):
    mu = jnp.mean(t, axis=0, keepdims=True)
    var = jnp.mean((t - mu) ** 2, axis=0, keepdims=True)
    return g * (t - mu) * lax.rsqrt(var + BN_EPS) + be


def _stage_a(N):
    def body(p_ref, x_ref, w1_ref, dis_ref, hp_ref):
        deg = p_ref[0:N, 0:1] + p_ref[N:2 * N, 0:1] + 1.0
        dis = lax.rsqrt(deg)
        dis_ref[...] = dis
        hw = jnp.dot(x_ref[...], w1_ref[...],
                     preferred_element_type=jnp.float32)
        hp_ref[...] = hw * dis
    return body


def _stage_b(N, relu_first):
    def body(p_ref, hp_ref, dis_ref, b_ref, g_ref, be_ref, w_ref, out_ref):
        dis = dis_ref[...]
        y = dis * (p_ref[0:N, :] + p_ref[N:2 * N, :] + hp_ref[...]) + b_ref[...]
        if relu_first:
            h = _bn(jnp.maximum(y, 0.0), g_ref[...], be_ref[...])
        else:
            h = jnp.maximum(_bn(y, g_ref[...], be_ref[...]), 0.0)
        hw = jnp.dot(h, w_ref[...], preferred_element_type=jnp.float32)
        out_ref[...] = hw * dis
    return body


def _stage_c(N):
    def body(agg_ref, hp_ref, dis_ref, b_ref, g_ref, be_ref,
             fc1w_ref, fc1b_ref, fc2w_ref, fc2b_ref, out_ref):
        dis = dis_ref[...]
        y = dis * (agg_ref[...] + hp_ref[...]) + b_ref[...]
        h = jnp.maximum(_bn(y, g_ref[...], be_ref[...]), 0.0)
        z = jnp.maximum(
            jnp.dot(h, fc1w_ref[...], preferred_element_type=jnp.float32)
            + fc1b_ref[...], 0.0)
        out_ref[...] = (jnp.dot(z, fc2w_ref[...],
                                preferred_element_type=jnp.float32)
                        + fc2b_ref[...])
    return body


def _tc_call(body, out_shapes, *args):
    return pl.pallas_call(
        body,
        out_shape=out_shapes,
    )(*args)


def kernel(x, edge_index, edge_attr,
           W1, b1, g1, be1, W2, b2, g2, be2, W3, b3, g3, be3,
           W4, b4, g4, be4, W5, b5, g5, be5,
           fc1_w, fc1_b, fc2_w, fc2_b):
    N = x.shape[0]
    E = edge_index.shape[1]
    EC = E // NW
    C = 80
    NCH = EC // C

    src3 = edge_index[0].reshape(NW, NCH, C)
    dst3 = edge_index[1].reshape(NW, NCH, C)
    w3 = edge_attr.reshape(NW, NCH, C)

    ones16 = jnp.ones((N, 16), jnp.float32)
    degp = _seg_sum_kernel(N, E, 16)(ones16, src3, dst3, w3)

    dis, hp = _tc_call(
        _stage_a(N),
        (jax.ShapeDtypeStruct((N, 1), jnp.float32),
         jax.ShapeDtypeStruct((N, W1.shape[1]), jnp.float32)),
        degp, x, W1)

    layer_params = [
        (b1, g1, be1, W2, True),
        (b2, g2, be2, W3, True),
        (b3, g3, be3, W4, True),
        (b4, g4, be4, W5, False),
    ]
    for (b, g, be, Wn, relu_first) in layer_params:
        D = hp.shape[1]
        p = _seg_sum_kernel(N, E, D)(hp, src3, dst3, w3)
        hp = _tc_call(
            _stage_b(N, relu_first),
            jax.ShapeDtypeStruct((N, Wn.shape[1]), jnp.float32),
            p, hp, dis, b.reshape(1, -1), g.reshape(1, -1),
            be.reshape(1, -1), Wn)

    D = hp.shape[1]
    DH = D // 2
    srcC = edge_index[0].reshape(NS, -1, 80)
    dstC = edge_index[1].reshape(NS, -1, 80)
    wC = edge_attr.reshape(NS, -1, 80)
    h2 = jnp.concatenate([hp[:, :DH], hp[:, DH:]], axis=0)
    p5 = _seg_sum_kernel(N, E, DH, True)(h2, srcC, dstC, wC)
    agg5 = jnp.concatenate([p5[:N], p5[N:]], axis=1)
    out = _tc_call(
        _stage_c(N),
        jax.ShapeDtypeStruct((N, 1), jnp.float32),
        agg5, hp, dis, b5.reshape(1, -1), g5.reshape(1, -1), be5.reshape(1, -1),
        fc1_w, fc1_b.reshape(1, -1), fc2_w, fc2_b.reshape(1, -1))
    return out.reshape(-1)
